# baseline (speedup 1.0000x reference)
import functools

import jax
import jax.numpy as jnp
from jax import lax
from jax.experimental import pallas as pl
from jax.experimental.pallas import tpu as pltpu
import jax.experimental.pallas.tpu_sc as plsc

N_TOK = 128
D_IN = 1024
V_DICT = 65536
TOPK = 64

ENC_BLK = 4096
DEC_BLK = 4096

_LANES = 16
_N16 = V_DICT // 16
_N128 = V_DICT // 128
_NWORK = 32
_RPW = N_TOK // _NWORK
_CHUNK_CAP = 128
_SUB_CAP = 512
_CAND_CAP = _SUB_CAP * 16
_ZBUF = 16384


def _encode_body(x_ref, w_ref, b_ref, out_ref, m16_ref):
    acts = lax.dot_general(
        x_ref[...], w_ref[...],
        dimension_numbers=(((1,), (1,)), ((), ())),
        preferred_element_type=jnp.float32,
    )
    acts = jnp.maximum(acts + b_ref[...], 0.0)
    out_ref[...] = acts
    m16_ref[...] = jnp.max(acts.reshape(N_TOK, ENC_BLK // 16, 16), axis=2)


def _encode(xc, enc_w, enc_b2d):
    return pl.pallas_call(
        _encode_body,
        grid=(V_DICT // ENC_BLK,),
        in_specs=[
            pl.BlockSpec((N_TOK, D_IN), lambda i: (0, 0)),
            pl.BlockSpec((ENC_BLK, D_IN), lambda i: (i, 0)),
            pl.BlockSpec((1, ENC_BLK), lambda i: (0, i)),
        ],
        out_specs=[
            pl.BlockSpec((N_TOK, ENC_BLK), lambda i: (0, i)),
            pl.BlockSpec((N_TOK, ENC_BLK // 16), lambda i: (0, i)),
        ],
        out_shape=[
            jax.ShapeDtypeStruct((N_TOK, V_DICT), jnp.float32),
            jax.ShapeDtypeStruct((N_TOK, _N16), jnp.float32),
        ],
    )(xc, enc_w, enc_b2d)


def _decode_body(sc_ref, w_ref, b_ref, out_ref):
    @pl.when(pl.program_id(0) == 0)
    def _():
        out_ref[...] = jnp.broadcast_to(b_ref[...], (N_TOK, D_IN))

    out_ref[...] += lax.dot_general(
        sc_ref[...], w_ref[...],
        dimension_numbers=(((1,), (1,)), ((), ())),
        preferred_element_type=jnp.float32,
    )


def _decode(sparse_code, dec_w, dec_bias2d):
    return pl.pallas_call(
        _decode_body,
        grid=(V_DICT // DEC_BLK,),
        in_specs=[
            pl.BlockSpec((N_TOK, DEC_BLK), lambda i: (0, i)),
            pl.BlockSpec((D_IN, DEC_BLK), lambda i: (0, i)),
            pl.BlockSpec((1, D_IN), lambda i: (0, 0)),
        ],
        out_specs=pl.BlockSpec((N_TOK, D_IN), lambda i: (0, 0)),
        out_shape=jax.ShapeDtypeStruct((N_TOK, D_IN), jnp.float32),
    )(sparse_code, dec_w, dec_bias2d)


def _allreduce(v, op):
    iota = jnp.arange(_LANES, dtype=jnp.int32)
    for sh in (8, 4, 2, 1):
        v = op(v, v.at[iota ^ sh].get(mode="promise_in_bounds"))
    return v


def _sc_select_body(acts_hbm, m16_hbm, sparse_hbm,
                    m16_v, m128_v, chunk_v, sub_v, idx_v, g_v,
                    cand_v, cand_i, topv_v, topi_v, topfv_v, topfi_v, zero_v,
                    zsem, gsem, ssem):
    cid = lax.axis_index("c")
    sid = lax.axis_index("s")
    wid = sid * 2 + cid
    iota = jnp.arange(_LANES, dtype=jnp.int32)
    zvec = jnp.zeros((_LANES,), jnp.float32)

    def _zero_init(i, carry):
        zero_v[pl.ds(i * _LANES, _LANES)] = zvec
        return carry

    lax.fori_loop(0, _ZBUF // _LANES, _zero_init, 0)

    def _row(rho, carry):
        r = wid * _RPW + rho

        zcopies = [
            pltpu.async_copy(
                zero_v, sparse_hbm.at[pl.ds(r * V_DICT + q * _ZBUF, _ZBUF)],
                zsem)
            for q in range(V_DICT // _ZBUF)
        ]

        pltpu.sync_copy(m16_hbm.at[r], m16_v)

        def _m128(v, carry):
            base = (v * _LANES + iota) * 8
            acc = plsc.load_gather(m16_v, [base])
            for j in range(1, 8):
                acc = jnp.maximum(acc, plsc.load_gather(m16_v, [base + j]))
            m128_v[pl.ds(v * _LANES, _LANES)] = acc
            return carry

        lax.fori_loop(0, _N128 // _LANES, _m128, 0)

        def _count_ge(t_f, ref, nvr):
            tb = jnp.full((_LANES,), t_f, jnp.float32)

            def _cnt(g, acc):
                v = ref[pl.ds(g * _LANES, _LANES)]
                return acc + jnp.where(v >= tb, 1, 0).astype(jnp.int32)

            acc = lax.fori_loop(0, nvr, _cnt, jnp.zeros((_LANES,), jnp.int32))
            return _allreduce(acc, jnp.add)[0]

        def _bsearch(ref, nvr):
            def _step(i, lohi):
                lo, hi = lohi
                mid = lo + ((hi - lo + 1) >> 1)
                mid_f = lax.bitcast_convert_type(mid, jnp.float32)
                feas = _count_ge(mid_f, ref, nvr) >= TOPK
                return (jnp.where(feas, mid, lo), jnp.where(feas, hi, mid - 1))

            lo, hi = lax.fori_loop(0, 31, _step,
                                   (jnp.int32(0), jnp.int32(0x7F800000)))
            return lax.bitcast_convert_type(lo, jnp.float32)

        theta_s = _bsearch(m128_v, _N128 // _LANES)
        theta = jnp.full((_LANES,), theta_s, jnp.float32)

        def _chunks(c, off):
            v = m128_v[pl.ds(c * _LANES, _LANES)]
            m = v >= theta
            offc = jnp.minimum(off, _CHUNK_CAP)
            plsc.store_compressed(chunk_v.at[pl.ds(offc, _LANES)],
                                  c * _LANES + iota, mask=m)
            return off + plsc.all_reduce_population_count(m)[0]

        n_chunk = lax.fori_loop(0, _N128 // _LANES, _chunks, jnp.int32(0))
        n_chunk = jnp.minimum(n_chunk, _CHUNK_CAP)

        k_lo = iota & 7
        is_hi = iota >= 8

        def _subs(p, off):
            h2 = chunk_v[pl.ds(2 * p, _LANES)]
            c0 = jnp.full((_LANES,), h2[0], jnp.int32)
            c1 = jnp.full((_LANES,), h2[1], jnp.int32)
            c = jnp.where(is_hi, c1, c0)
            valid = jnp.logical_or(jnp.logical_not(is_hi), 2 * p + 1 < n_chunk)
            sidx = (c * 8 + k_lo) & (_N16 - 1)
            mv = plsc.load_gather(m16_v, [sidx])
            m = jnp.logical_and(jnp.logical_and(mv >= theta, valid),
                                off < _SUB_CAP)
            offc = jnp.minimum(off, _SUB_CAP)
            plsc.store_compressed(sub_v.at[pl.ds(offc, _LANES)], sidx, mask=m)
            return off + plsc.all_reduce_population_count(m)[0]

        npairs = lax.div(n_chunk + 1, jnp.int32(2))
        n_sub = lax.fori_loop(0, npairs, _subs, jnp.int32(0))
        n_sub = jnp.minimum(n_sub, _SUB_CAP)

        base_g = r * V_DICT

        def _mkidx(j, carry):
            sj = sub_v[pl.ds(j, _LANES)]
            s0 = jnp.full((_LANES,), sj[0], jnp.int32)
            idx_v[pl.ds(j * _LANES, _LANES)] = base_g + s0 * 16 + iota
            return carry

        lax.fori_loop(0, n_sub, _mkidx, 0)

        n_el = n_sub * _LANES
        n_dma = lax.div(n_el + 127, jnp.int32(128))

        def _pad(j, carry):
            idx_v[pl.ds(n_el + j * _LANES, _LANES)] = jnp.full(
                (_LANES,), base_g, jnp.int32)
            return carry

        lax.fori_loop(0, (n_dma * 128 - n_el) // _LANES + 1, _pad, 0)

        def _fire(j, carry):
            pltpu.async_copy(acts_hbm.at[idx_v.at[pl.ds(j * 128, 128)]],
                             g_v.at[pl.ds(j * 128, 128)], gsem)
            return carry

        lax.fori_loop(0, n_dma, _fire, 0)

        def _drain(j, carry):
            pltpu.make_async_copy(
                acts_hbm.at[idx_v.at[pl.ds(j * 128, 128)]],
                g_v.at[pl.ds(j * 128, 128)], gsem).wait()
            return carry

        lax.fori_loop(0, n_dma, _drain, 0)

        def _filter(j, off):
            v = g_v[pl.ds(j * _LANES, _LANES)]
            vi = idx_v[pl.ds(j * _LANES, _LANES)]
            m = v >= theta
            offc = jnp.minimum(off, _CAND_CAP)
            plsc.store_compressed(cand_v.at[pl.ds(offc, _LANES)], v, mask=m)
            plsc.store_compressed(cand_i.at[pl.ds(offc, _LANES)], vi, mask=m)
            return off + plsc.all_reduce_population_count(m)[0]

        n_cand = lax.fori_loop(0, n_sub, _filter, jnp.int32(0))
        n_cand = jnp.minimum(n_cand, _CAND_CAP)
        cand_v[pl.ds(n_cand, _LANES)] = jnp.full((_LANES,), -1.0, jnp.float32)
        cand_i[pl.ds(n_cand, _LANES)] = jnp.full((_LANES,), 0x3FFFFFFF,
                                                 jnp.int32)
        nv = lax.div(n_cand + 15, jnp.int32(_LANES))

        tau_s = _bsearch(cand_v, nv)
        tau = jnp.full((_LANES,), tau_s, jnp.float32)

        def _emit_gt(j, off):
            v = cand_v[pl.ds(j * _LANES, _LANES)]
            vi = cand_i[pl.ds(j * _LANES, _LANES)]
            m = v > tau
            offc = jnp.minimum(off, TOPK - 1)
            plsc.store_compressed(topv_v.at[pl.ds(offc, _LANES)], v, mask=m)
            plsc.store_compressed(topi_v.at[pl.ds(offc, _LANES)], vi, mask=m)
            return off + plsc.all_reduce_population_count(m)[0]

        m1 = lax.fori_loop(0, nv, _emit_gt, jnp.int32(0))

        def _fill(t, carry):
            def _scan(j, acc):
                v = cand_v[pl.ds(j * _LANES, _LANES)]
                vi = cand_i[pl.ds(j * _LANES, _LANES)]
                m = v == tau
                big = jnp.full((_LANES,), 0x7FFFFFFF, jnp.int32)
                return jnp.minimum(acc, jnp.where(m, vi, big))

            best = lax.fori_loop(
                0, nv, _scan, jnp.full((_LANES,), 0x7FFFFFFF, jnp.int32))
            besti = _allreduce(best, jnp.minimum)
            tsplat = jnp.full((_LANES,), m1 + t, jnp.int32)
            lane0 = iota == 0
            plsc.store_scatter(topv_v, [tsplat], tau, mask=lane0)
            plsc.store_scatter(topi_v, [tsplat], besti, mask=lane0)

            def _kill(j, carry2):
                v = cand_v[pl.ds(j * _LANES, _LANES)]
                vi = cand_i[pl.ds(j * _LANES, _LANES)]
                hit = jnp.logical_and(v == tau, vi == besti)
                cand_v[pl.ds(j * _LANES, _LANES)] = jnp.where(hit, -1.0, v)
                return carry2

            lax.fori_loop(0, nv, _kill, 0)
            return carry

        lax.fori_loop(0, TOPK - m1, _fill, 0)

        for w in range(TOPK // _LANES):
            topfv_v[pl.ds(w * _LANES, _LANES)] = topv_v[
                pl.ds(w * _LANES, _LANES)]
            topfi_v[pl.ds(w * _LANES, _LANES)] = topi_v[
                pl.ds(w * _LANES, _LANES)]

        for zc in zcopies:
            zc.wait()
        pltpu.async_copy(topfv_v, sparse_hbm.at[topfi_v], ssem).wait()
        return carry

    lax.fori_loop(0, _RPW, _row, 0)


def _make_sc_select(interpret=False):
    return pl.kernel(
        _sc_select_body,
        out_type=jax.ShapeDtypeStruct((N_TOK * V_DICT,), jnp.float32),
        mesh=plsc.VectorSubcoreMesh(core_axis_name="c", subcore_axis_name="s",
                                    num_cores=2, num_subcores=16),
        compiler_params=pltpu.CompilerParams(needs_layout_passes=False),
        scratch_types=[
            pltpu.VMEM((_N16,), jnp.float32),
            pltpu.VMEM((_N128,), jnp.float32),
            pltpu.VMEM((_CHUNK_CAP + 2 * _LANES,), jnp.int32),
            pltpu.VMEM((_SUB_CAP + 2 * _LANES,), jnp.int32),
            pltpu.VMEM((_CAND_CAP + 256,), jnp.int32),
            pltpu.VMEM((_CAND_CAP + 256,), jnp.float32),
            pltpu.VMEM((_CAND_CAP + 2 * _LANES,), jnp.float32),
            pltpu.VMEM((_CAND_CAP + 2 * _LANES,), jnp.int32),
            pltpu.VMEM((TOPK + _LANES,), jnp.float32),
            pltpu.VMEM((TOPK + _LANES,), jnp.int32),
            pltpu.VMEM((TOPK,), jnp.float32),
            pltpu.VMEM((TOPK,), jnp.int32),
            pltpu.VMEM((_ZBUF,), jnp.float32),
            pltpu.SemaphoreType.DMA,
            pltpu.SemaphoreType.DMA,
            pltpu.SemaphoreType.DMA,
        ],
        interpret=interpret,
    )


_sc_select = _make_sc_select()


@jax.jit
def kernel(x, enc_w, enc_b, dec_w, dec_bias):
    xc = x - dec_bias
    acts, m16 = _encode(xc, enc_w, enc_b.reshape(1, V_DICT))
    sparse_flat = _sc_select(acts.reshape(N_TOK * V_DICT), m16)
    sparse_code = sparse_flat.reshape(N_TOK, V_DICT)
    recon = _decode(sparse_code, dec_w, dec_bias.reshape(1, D_IN))
    return (recon, sparse_code)


# speedup vs baseline: 1.0898x; 1.0898x over previous
import functools

import jax
import jax.numpy as jnp
from jax import lax
from jax.experimental import pallas as pl
from jax.experimental.pallas import tpu as pltpu
import jax.experimental.pallas.tpu_sc as plsc

N_TOK = 128
D_IN = 1024
V_DICT = 65536
TOPK = 64

ENC_BLK = 4096
DEC_BLK = 4096

_LANES = 16
_N16 = V_DICT // 16
_N128 = V_DICT // 128
_NWORK = 32
_RPW = N_TOK // _NWORK
_CHUNK_CAP = 128
_SUB_CAP = 512
_CAND_CAP = _SUB_CAP * 16
_ZBUF = 16384


def _encode_body(x_ref, w_ref, b_ref, out_ref, m16_ref):
    acts = lax.dot_general(
        x_ref[...], w_ref[...],
        dimension_numbers=(((1,), (1,)), ((), ())),
        preferred_element_type=jnp.float32,
    )
    acts = jnp.maximum(acts + b_ref[...], 0.0)
    out_ref[...] = acts
    m16_ref[...] = jnp.max(acts.reshape(N_TOK, ENC_BLK // 16, 16), axis=2)


def _encode(xc, enc_w, enc_b2d):
    return pl.pallas_call(
        _encode_body,
        grid=(V_DICT // ENC_BLK,),
        in_specs=[
            pl.BlockSpec((N_TOK, D_IN), lambda i: (0, 0)),
            pl.BlockSpec((ENC_BLK, D_IN), lambda i: (i, 0)),
            pl.BlockSpec((1, ENC_BLK), lambda i: (0, i)),
        ],
        out_specs=[
            pl.BlockSpec((N_TOK, ENC_BLK), lambda i: (0, i)),
            pl.BlockSpec((N_TOK, ENC_BLK // 16), lambda i: (0, i)),
        ],
        out_shape=[
            jax.ShapeDtypeStruct((N_TOK, V_DICT), jnp.float32),
            jax.ShapeDtypeStruct((N_TOK, _N16), jnp.float32),
        ],
    )(xc, enc_w, enc_b2d)


def _decode_body(sc_ref, w_ref, b_ref, out_ref):
    @pl.when(pl.program_id(0) == 0)
    def _():
        out_ref[...] = jnp.broadcast_to(b_ref[...], (N_TOK, D_IN))

    out_ref[...] += lax.dot_general(
        sc_ref[...], w_ref[...],
        dimension_numbers=(((1,), (1,)), ((), ())),
        preferred_element_type=jnp.float32,
    )


def _decode(sparse_code, dec_w, dec_bias2d):
    return pl.pallas_call(
        _decode_body,
        grid=(V_DICT // DEC_BLK,),
        in_specs=[
            pl.BlockSpec((N_TOK, DEC_BLK), lambda i: (0, i)),
            pl.BlockSpec((D_IN, DEC_BLK), lambda i: (0, i)),
            pl.BlockSpec((1, D_IN), lambda i: (0, 0)),
        ],
        out_specs=pl.BlockSpec((N_TOK, D_IN), lambda i: (0, 0)),
        out_shape=jax.ShapeDtypeStruct((N_TOK, D_IN), jnp.float32),
    )(sparse_code, dec_w, dec_bias2d)


def _allreduce(v, op):
    iota = jnp.arange(_LANES, dtype=jnp.int32)
    for sh in (8, 4, 2, 1):
        v = op(v, v.at[iota ^ sh].get(mode="promise_in_bounds"))
    return v


def _sc_select_body(acts_hbm, m16_hbm, sparse_hbm,
                    row_v, m16_v, m128_v, chunk_v, sub_v,
                    cand_v, cand_i, topv_v, topi_v, topfv_v, topfi_v, zero_v,
                    zsem, rsem, ssem):
    cid = lax.axis_index("c")
    sid = lax.axis_index("s")
    wid = sid * 2 + cid
    iota = jnp.arange(_LANES, dtype=jnp.int32)
    zvec = jnp.zeros((_LANES,), jnp.float32)

    def _zero_init(i, carry):
        zero_v[pl.ds(i * _LANES, _LANES)] = zvec
        return carry

    lax.fori_loop(0, _ZBUF // _LANES, _zero_init, 0)

    def _row(rho, carry):
        r = wid * _RPW + rho

        zcopies = [
            pltpu.async_copy(
                zero_v, sparse_hbm.at[pl.ds(r * V_DICT + q * _ZBUF, _ZBUF)],
                zsem)
            for q in range(V_DICT // _ZBUF)
        ]

        row_copy = pltpu.async_copy(acts_hbm.at[r, pl.ds(0, V_DICT)],
                                    row_v, rsem)
        pltpu.sync_copy(m16_hbm.at[r], m16_v)

        def _m128(v, carry):
            base = (v * _LANES + iota) * 8
            acc = plsc.load_gather(m16_v, [base])
            for j in range(1, 8):
                acc = jnp.maximum(acc, plsc.load_gather(m16_v, [base + j]))
            m128_v[pl.ds(v * _LANES, _LANES)] = acc
            return carry

        lax.fori_loop(0, _N128 // _LANES, _m128, 0)

        def _count_ge(t_f, ref, nvr):
            tb = jnp.full((_LANES,), t_f, jnp.float32)

            def _cnt(g, acc):
                v = ref[pl.ds(g * _LANES, _LANES)]
                return acc + jnp.where(v >= tb, 1, 0).astype(jnp.int32)

            acc = lax.fori_loop(0, nvr, _cnt, jnp.zeros((_LANES,), jnp.int32))
            return _allreduce(acc, jnp.add)[0]

        def _bsearch(ref, nvr):
            def _step(i, lohi):
                lo, hi = lohi
                mid = lo + ((hi - lo + 1) >> 1)
                mid_f = lax.bitcast_convert_type(mid, jnp.float32)
                feas = _count_ge(mid_f, ref, nvr) >= TOPK
                return (jnp.where(feas, mid, lo), jnp.where(feas, hi, mid - 1))

            lo, hi = lax.fori_loop(0, 31, _step,
                                   (jnp.int32(0), jnp.int32(0x7F800000)))
            return lax.bitcast_convert_type(lo, jnp.float32)

        theta_s = _bsearch(m128_v, _N128 // _LANES)
        theta = jnp.full((_LANES,), theta_s, jnp.float32)

        def _chunks(c, off):
            v = m128_v[pl.ds(c * _LANES, _LANES)]
            m = v >= theta
            offc = jnp.minimum(off, _CHUNK_CAP)
            plsc.store_compressed(chunk_v.at[pl.ds(offc, _LANES)],
                                  c * _LANES + iota, mask=m)
            return off + plsc.all_reduce_population_count(m)[0]

        n_chunk = lax.fori_loop(0, _N128 // _LANES, _chunks, jnp.int32(0))
        n_chunk = jnp.minimum(n_chunk, _CHUNK_CAP)

        k_lo = iota & 7
        is_hi = iota >= 8

        def _subs(p, off):
            h2 = chunk_v[pl.ds(2 * p, _LANES)]
            c0 = jnp.full((_LANES,), h2[0], jnp.int32)
            c1 = jnp.full((_LANES,), h2[1], jnp.int32)
            c = jnp.where(is_hi, c1, c0)
            valid = jnp.logical_or(jnp.logical_not(is_hi), 2 * p + 1 < n_chunk)
            sidx = (c * 8 + k_lo) & (_N16 - 1)
            mv = plsc.load_gather(m16_v, [sidx])
            m = jnp.logical_and(jnp.logical_and(mv >= theta, valid),
                                off < _SUB_CAP)
            offc = jnp.minimum(off, _SUB_CAP)
            plsc.store_compressed(sub_v.at[pl.ds(offc, _LANES)], sidx, mask=m)
            return off + plsc.all_reduce_population_count(m)[0]

        npairs = lax.div(n_chunk + 1, jnp.int32(2))
        n_sub = lax.fori_loop(0, npairs, _subs, jnp.int32(0))
        n_sub = jnp.minimum(n_sub, _SUB_CAP)

        base_g = r * V_DICT
        row_copy.wait()

        def _filter(j, off):
            sj = sub_v[pl.ds(j, _LANES)]
            s0 = sj[0]
            v = row_v[pl.ds(s0 * 16, _LANES)]
            vi = base_g + s0 * 16 + iota
            m = v >= theta
            offc = jnp.minimum(off, _CAND_CAP)
            plsc.store_compressed(cand_v.at[pl.ds(offc, _LANES)], v, mask=m)
            plsc.store_compressed(cand_i.at[pl.ds(offc, _LANES)], vi, mask=m)
            return off + plsc.all_reduce_population_count(m)[0]

        n_cand = lax.fori_loop(0, n_sub, _filter, jnp.int32(0))
        n_cand = jnp.minimum(n_cand, _CAND_CAP)
        cand_v[pl.ds(n_cand, _LANES)] = jnp.full((_LANES,), -1.0, jnp.float32)
        cand_i[pl.ds(n_cand, _LANES)] = jnp.full((_LANES,), 0x3FFFFFFF,
                                                 jnp.int32)
        nv = lax.div(n_cand + 15, jnp.int32(_LANES))

        tau_s = _bsearch(cand_v, nv)
        tau = jnp.full((_LANES,), tau_s, jnp.float32)

        def _emit_gt(j, off):
            v = cand_v[pl.ds(j * _LANES, _LANES)]
            vi = cand_i[pl.ds(j * _LANES, _LANES)]
            m = v > tau
            offc = jnp.minimum(off, TOPK - 1)
            plsc.store_compressed(topv_v.at[pl.ds(offc, _LANES)], v, mask=m)
            plsc.store_compressed(topi_v.at[pl.ds(offc, _LANES)], vi, mask=m)
            return off + plsc.all_reduce_population_count(m)[0]

        m1 = lax.fori_loop(0, nv, _emit_gt, jnp.int32(0))

        def _fill(t, carry):
            def _scan(j, acc):
                v = cand_v[pl.ds(j * _LANES, _LANES)]
                vi = cand_i[pl.ds(j * _LANES, _LANES)]
                m = v == tau
                big = jnp.full((_LANES,), 0x7FFFFFFF, jnp.int32)
                return jnp.minimum(acc, jnp.where(m, vi, big))

            best = lax.fori_loop(
                0, nv, _scan, jnp.full((_LANES,), 0x7FFFFFFF, jnp.int32))
            besti = _allreduce(best, jnp.minimum)
            tsplat = jnp.full((_LANES,), m1 + t, jnp.int32)
            lane0 = iota == 0
            plsc.store_scatter(topv_v, [tsplat], tau, mask=lane0)
            plsc.store_scatter(topi_v, [tsplat], besti, mask=lane0)

            def _kill(j, carry2):
                v = cand_v[pl.ds(j * _LANES, _LANES)]
                vi = cand_i[pl.ds(j * _LANES, _LANES)]
                hit = jnp.logical_and(v == tau, vi == besti)
                cand_v[pl.ds(j * _LANES, _LANES)] = jnp.where(hit, -1.0, v)
                return carry2

            lax.fori_loop(0, nv, _kill, 0)
            return carry

        lax.fori_loop(0, TOPK - m1, _fill, 0)

        for w in range(TOPK // _LANES):
            topfv_v[pl.ds(w * _LANES, _LANES)] = topv_v[
                pl.ds(w * _LANES, _LANES)]
            topfi_v[pl.ds(w * _LANES, _LANES)] = topi_v[
                pl.ds(w * _LANES, _LANES)]

        for zc in zcopies:
            zc.wait()
        pltpu.async_copy(topfv_v, sparse_hbm.at[topfi_v], ssem).wait()
        return carry

    lax.fori_loop(0, _RPW, _row, 0)


def _make_sc_select(interpret=False):
    return pl.kernel(
        _sc_select_body,
        out_type=jax.ShapeDtypeStruct((N_TOK * V_DICT,), jnp.float32),
        mesh=plsc.VectorSubcoreMesh(core_axis_name="c", subcore_axis_name="s",
                                    num_cores=2, num_subcores=16),
        compiler_params=pltpu.CompilerParams(needs_layout_passes=False),
        scratch_types=[
            pltpu.VMEM((V_DICT,), jnp.float32),
            pltpu.VMEM((_N16,), jnp.float32),
            pltpu.VMEM((_N128,), jnp.float32),
            pltpu.VMEM((_CHUNK_CAP + 2 * _LANES,), jnp.int32),
            pltpu.VMEM((_SUB_CAP + 2 * _LANES,), jnp.int32),
            pltpu.VMEM((_CAND_CAP + 2 * _LANES,), jnp.float32),
            pltpu.VMEM((_CAND_CAP + 2 * _LANES,), jnp.int32),
            pltpu.VMEM((TOPK + _LANES,), jnp.float32),
            pltpu.VMEM((TOPK + _LANES,), jnp.int32),
            pltpu.VMEM((TOPK,), jnp.float32),
            pltpu.VMEM((TOPK,), jnp.int32),
            pltpu.VMEM((_ZBUF,), jnp.float32),
            pltpu.SemaphoreType.DMA,
            pltpu.SemaphoreType.DMA,
            pltpu.SemaphoreType.DMA,
        ],
        interpret=interpret,
    )


_sc_select = _make_sc_select()


@jax.jit
def kernel(x, enc_w, enc_b, dec_w, dec_bias):
    xc = x - dec_bias
    acts, m16 = _encode(xc, enc_w, enc_b.reshape(1, V_DICT))
    sparse_flat = _sc_select(acts, m16)
    sparse_code = sparse_flat.reshape(N_TOK, V_DICT)
    recon = _decode(sparse_code, dec_w, dec_bias.reshape(1, D_IN))
    return (recon, sparse_code)


# speedup vs baseline: 1.1885x; 1.0905x over previous
import functools

import jax
import jax.numpy as jnp
from jax import lax
from jax.experimental import pallas as pl
from jax.experimental.pallas import tpu as pltpu
import jax.experimental.pallas.tpu_sc as plsc

N_TOK = 128
D_IN = 1024
V_DICT = 65536
TOPK = 64

ENC_BLK = 4096
DEC_BLK = 4096

_LANES = 16
_NVREG = V_DICT // _LANES
_L1 = _NVREG // 8
_L2 = _L1 // 8
_NWORK = 32
_RPW = N_TOK // _NWORK
_CHUNK_CAP = 128
_SUB_CAP = 512
_CAND_CAP = 4096
_ZBUF = 16384


def _encode_body(x_ref, w_ref, b_ref, out_ref):
    acts = lax.dot_general(
        x_ref[...], w_ref[...],
        dimension_numbers=(((1,), (1,)), ((), ())),
        preferred_element_type=jnp.float32,
    )
    out_ref[...] = jnp.maximum(acts + b_ref[...], 0.0)


def _encode(xc, enc_w, enc_b2d):
    return pl.pallas_call(
        _encode_body,
        grid=(V_DICT // ENC_BLK,),
        in_specs=[
            pl.BlockSpec((N_TOK, D_IN), lambda i: (0, 0)),
            pl.BlockSpec((ENC_BLK, D_IN), lambda i: (i, 0)),
            pl.BlockSpec((1, ENC_BLK), lambda i: (0, i)),
        ],
        out_specs=pl.BlockSpec((N_TOK, ENC_BLK), lambda i: (0, i)),
        out_shape=jax.ShapeDtypeStruct((N_TOK, V_DICT), jnp.float32),
    )(xc, enc_w, enc_b2d)


def _decode_body(sc_ref, w_ref, b_ref, out_ref):
    @pl.when(pl.program_id(0) == 0)
    def _():
        out_ref[...] = jnp.broadcast_to(b_ref[...], (N_TOK, D_IN))

    out_ref[...] += lax.dot_general(
        sc_ref[...], w_ref[...],
        dimension_numbers=(((1,), (1,)), ((), ())),
        preferred_element_type=jnp.float32,
    )


def _decode(sparse_code, dec_w, dec_bias2d):
    return pl.pallas_call(
        _decode_body,
        grid=(V_DICT // DEC_BLK,),
        in_specs=[
            pl.BlockSpec((N_TOK, DEC_BLK), lambda i: (0, i)),
            pl.BlockSpec((D_IN, DEC_BLK), lambda i: (0, i)),
            pl.BlockSpec((1, D_IN), lambda i: (0, 0)),
        ],
        out_specs=pl.BlockSpec((N_TOK, D_IN), lambda i: (0, 0)),
        out_shape=jax.ShapeDtypeStruct((N_TOK, D_IN), jnp.float32),
    )(sparse_code, dec_w, dec_bias2d)


def _allreduce(v, op):
    iota = jnp.arange(_LANES, dtype=jnp.int32)
    for sh in (8, 4, 2, 1):
        v = op(v, v.at[iota ^ sh].get(mode="promise_in_bounds"))
    return v


def _sc_select_body(acts_hbm, sparse_hbm,
                    row_v, l1_v, l2_v, chunk_v, sub_v,
                    cand_v, cand_i, topv_v, topi_v, topfv_v, topfi_v, zero_v,
                    zsem, rsem, ssem):
    cid = lax.axis_index("c")
    sid = lax.axis_index("s")
    wid = sid * 2 + cid
    iota = jnp.arange(_LANES, dtype=jnp.int32)
    zvec = jnp.zeros((_LANES,), jnp.float32)

    def _zero_init(i, carry):
        zero_v[pl.ds(i * _LANES, _LANES)] = zvec
        return carry

    lax.fori_loop(0, _ZBUF // _LANES, _zero_init, 0)

    def _row(rho, carry):
        r = wid * _RPW + rho

        zcopies = [
            pltpu.async_copy(
                zero_v, sparse_hbm.at[pl.ds(r * V_DICT + q * _ZBUF, _ZBUF)],
                zsem)
            for q in range(V_DICT // _ZBUF)
        ]

        _Q = V_DICT // 4
        qcopies = [
            pltpu.async_copy(acts_hbm.at[r, pl.ds(q * _Q, _Q)],
                             row_v.at[pl.ds(q * _Q, _Q)], rsem)
            for q in range(4)
        ]

        def _l1(c, carry):
            base = c * 128
            acc = row_v[pl.ds(base, _LANES)]
            for j in range(1, 8):
                acc = jnp.maximum(acc, row_v[pl.ds(base + j * _LANES, _LANES)])
            l1_v[pl.ds(c * _LANES, _LANES)] = acc
            return carry

        for q in range(4):
            qcopies[q].wait()
            lax.fori_loop(q * (_L1 // 4), (q + 1) * (_L1 // 4), _l1, 0)

        def _l2(g, carry):
            base = g * 128
            acc = l1_v[pl.ds(base, _LANES)]
            for j in range(1, 8):
                acc = jnp.maximum(acc, l1_v[pl.ds(base + j * _LANES, _LANES)])
            l2_v[pl.ds(g * _LANES, _LANES)] = acc
            return carry

        lax.fori_loop(0, _L2, _l2, 0)

        def _count_ge(t_f, ref, nvr):
            tb = jnp.full((_LANES,), t_f, jnp.float32)

            def _cnt(g, acc):
                v = ref[pl.ds(g * _LANES, _LANES)]
                return acc + jnp.where(v >= tb, 1, 0).astype(jnp.int32)

            acc = lax.fori_loop(0, nvr, _cnt, jnp.zeros((_LANES,), jnp.int32))
            return _allreduce(acc, jnp.add)[0]

        def _bsearch(ref, nvr):
            def _step(i, lohi):
                lo, hi = lohi
                mid = lo + ((hi - lo + 1) >> 1)
                mid_f = lax.bitcast_convert_type(mid, jnp.float32)
                feas = _count_ge(mid_f, ref, nvr) >= TOPK
                return (jnp.where(feas, mid, lo), jnp.where(feas, hi, mid - 1))

            lo, hi = lax.fori_loop(0, 31, _step,
                                   (jnp.int32(0), jnp.int32(0x7F800000)))
            return lax.bitcast_convert_type(lo, jnp.float32)

        theta_s = _bsearch(l2_v, _L2)
        theta = jnp.full((_LANES,), theta_s, jnp.float32)

        def _chunks(g, off):
            v = l2_v[pl.ds(g * _LANES, _LANES)]
            m = v >= theta
            offc = jnp.minimum(off, _CHUNK_CAP)
            plsc.store_compressed(chunk_v.at[pl.ds(offc, _LANES)],
                                  g * _LANES + iota, mask=m)
            return off + plsc.all_reduce_population_count(m)[0]

        n_chunk = lax.fori_loop(0, _L2, _chunks, jnp.int32(0))
        n_chunk = jnp.minimum(n_chunk, _CHUNK_CAP)

        k_lo = iota & 7
        is_hi = iota >= 8

        def _subs(p, off):
            h2 = chunk_v[pl.ds(2 * p, _LANES)]
            g0 = jnp.full((_LANES,), h2[0], jnp.int32)
            g1 = jnp.full((_LANES,), h2[1], jnp.int32)
            g = jnp.where(is_hi, g1, g0)
            valid = jnp.logical_or(jnp.logical_not(is_hi), 2 * p + 1 < n_chunk)
            gid1 = (((g >> 4) * 8 + k_lo) * 16 + (g & 15)) & (_L1 * 16 - 1)
            mv = plsc.load_gather(l1_v, [gid1])
            m = jnp.logical_and(jnp.logical_and(mv >= theta, valid),
                                off < _SUB_CAP)
            offc = jnp.minimum(off, _SUB_CAP)
            plsc.store_compressed(sub_v.at[pl.ds(offc, _LANES)], gid1, mask=m)
            return off + plsc.all_reduce_population_count(m)[0]

        npairs = lax.div(n_chunk + 1, jnp.int32(2))
        n_sub = lax.fori_loop(0, npairs, _subs, jnp.int32(0))
        n_sub = jnp.minimum(n_sub, _SUB_CAP)

        base_g = r * V_DICT

        def _filter(p, off):
            h2 = sub_v[pl.ds(2 * p, _LANES)]
            g0 = jnp.full((_LANES,), h2[0], jnp.int32)
            g1 = jnp.full((_LANES,), h2[1], jnp.int32)
            g = jnp.where(is_hi, g1, g0)
            valid = jnp.logical_or(jnp.logical_not(is_hi), 2 * p + 1 < n_sub)
            eidx = (((g >> 4) << 7) + (g & 15) + (k_lo << 4)) & (V_DICT - 1)
            v = plsc.load_gather(row_v, [eidx])
            m = jnp.logical_and(jnp.logical_and(v >= theta, valid),
                                off < _CAND_CAP)
            offc = jnp.minimum(off, _CAND_CAP)
            plsc.store_compressed(cand_v.at[pl.ds(offc, _LANES)], v, mask=m)
            plsc.store_compressed(cand_i.at[pl.ds(offc, _LANES)],
                                  base_g + eidx, mask=m)
            return off + plsc.all_reduce_population_count(m)[0]

        npairs2 = lax.div(n_sub + 1, jnp.int32(2))
        n_cand = lax.fori_loop(0, npairs2, _filter, jnp.int32(0))
        n_cand = jnp.minimum(n_cand, _CAND_CAP)
        cand_v[pl.ds(n_cand, _LANES)] = jnp.full((_LANES,), -1.0, jnp.float32)
        cand_i[pl.ds(n_cand, _LANES)] = jnp.full((_LANES,), 0x3FFFFFFF,
                                                 jnp.int32)
        nv = lax.div(n_cand + 15, jnp.int32(_LANES))

        tau_s = _bsearch(cand_v, nv)
        tau = jnp.full((_LANES,), tau_s, jnp.float32)

        def _emit_gt(j, off):
            v = cand_v[pl.ds(j * _LANES, _LANES)]
            vi = cand_i[pl.ds(j * _LANES, _LANES)]
            m = v > tau
            offc = jnp.minimum(off, TOPK - 1)
            plsc.store_compressed(topv_v.at[pl.ds(offc, _LANES)], v, mask=m)
            plsc.store_compressed(topi_v.at[pl.ds(offc, _LANES)], vi, mask=m)
            return off + plsc.all_reduce_population_count(m)[0]

        m1 = lax.fori_loop(0, nv, _emit_gt, jnp.int32(0))

        def _fill(t, carry):
            def _scan(j, acc):
                v = cand_v[pl.ds(j * _LANES, _LANES)]
                vi = cand_i[pl.ds(j * _LANES, _LANES)]
                m = v == tau
                big = jnp.full((_LANES,), 0x7FFFFFFF, jnp.int32)
                return jnp.minimum(acc, jnp.where(m, vi, big))

            best = lax.fori_loop(
                0, nv, _scan, jnp.full((_LANES,), 0x7FFFFFFF, jnp.int32))
            besti = _allreduce(best, jnp.minimum)
            tsplat = jnp.full((_LANES,), m1 + t, jnp.int32)
            lane0 = iota == 0
            plsc.store_scatter(topv_v, [tsplat], tau, mask=lane0)
            plsc.store_scatter(topi_v, [tsplat], besti, mask=lane0)

            def _kill(j, carry2):
                v = cand_v[pl.ds(j * _LANES, _LANES)]
                vi = cand_i[pl.ds(j * _LANES, _LANES)]
                hit = jnp.logical_and(v == tau, vi == besti)
                cand_v[pl.ds(j * _LANES, _LANES)] = jnp.where(hit, -1.0, v)
                return carry2

            lax.fori_loop(0, nv, _kill, 0)
            return carry

        lax.fori_loop(0, TOPK - m1, _fill, 0)

        for w in range(TOPK // _LANES):
            topfv_v[pl.ds(w * _LANES, _LANES)] = topv_v[
                pl.ds(w * _LANES, _LANES)]
            topfi_v[pl.ds(w * _LANES, _LANES)] = topi_v[
                pl.ds(w * _LANES, _LANES)]

        for zc in zcopies:
            zc.wait()
        pltpu.async_copy(topfv_v, sparse_hbm.at[topfi_v], ssem).wait()
        return carry

    lax.fori_loop(0, _RPW, _row, 0)


def _make_sc_select(interpret=False):
    return pl.kernel(
        _sc_select_body,
        out_type=jax.ShapeDtypeStruct((N_TOK * V_DICT,), jnp.float32),
        mesh=plsc.VectorSubcoreMesh(core_axis_name="c", subcore_axis_name="s",
                                    num_cores=2, num_subcores=16),
        compiler_params=pltpu.CompilerParams(needs_layout_passes=False),
        scratch_types=[
            pltpu.VMEM((V_DICT,), jnp.float32),
            pltpu.VMEM((_L1 * _LANES,), jnp.float32),
            pltpu.VMEM((_L2 * _LANES,), jnp.float32),
            pltpu.VMEM((_CHUNK_CAP + 2 * _LANES,), jnp.int32),
            pltpu.VMEM((_SUB_CAP + 2 * _LANES,), jnp.int32),
            pltpu.VMEM((_CAND_CAP + 2 * _LANES,), jnp.float32),
            pltpu.VMEM((_CAND_CAP + 2 * _LANES,), jnp.int32),
            pltpu.VMEM((TOPK + _LANES,), jnp.float32),
            pltpu.VMEM((TOPK + _LANES,), jnp.int32),
            pltpu.VMEM((TOPK,), jnp.float32),
            pltpu.VMEM((TOPK,), jnp.int32),
            pltpu.VMEM((_ZBUF,), jnp.float32),
            pltpu.SemaphoreType.DMA,
            pltpu.SemaphoreType.DMA,
            pltpu.SemaphoreType.DMA,
        ],
        interpret=interpret,
    )


_sc_select = _make_sc_select()


@jax.jit
def kernel(x, enc_w, enc_b, dec_w, dec_bias):
    xc = x - dec_bias
    acts = _encode(xc, enc_w, enc_b.reshape(1, V_DICT))
    sparse_flat = _sc_select(acts)
    sparse_code = sparse_flat.reshape(N_TOK, V_DICT)
    recon = _decode(sparse_code, dec_w, dec_bias.reshape(1, D_IN))
    return (recon, sparse_code)
